# baseline (device time: 10260 ns/iter reference)
import jax
import jax.numpy as jnp
from jax import lax
from jax.experimental import pallas as pl
from jax.experimental.pallas import tpu as pltpu

N_DEV = 8
N_CHUNKS = 8


def kernel(x):
    m_per, n = x.shape
    chunk = m_per // N_CHUNKS

    def body(x_hbm, out_ref, buf_ref, gather_ref, copy_sems, send_sems,
             recv_sems):
        my_pos = lax.axis_index("i")

        barrier_sem = pltpu.get_barrier_semaphore()
        for k in range(1, N_DEV):
            pl.semaphore_signal(
                barrier_sem, inc=1,
                device_id=((my_pos + k) % N_DEV,),
                device_id_type=pl.DeviceIdType.MESH,
            )

        def copy_in(c):
            return pltpu.make_async_copy(
                x_hbm.at[pl.ds(c * chunk, chunk)],
                buf_ref.at[c % 2],
                copy_sems.at[c % 2],
            )

        copy_in(0).start()
        acc = None
        for c in range(N_CHUNKS):
            if c + 1 < N_CHUNKS:
                copy_in(c + 1).start()
            copy_in(c).wait()
            part = jnp.max(buf_ref[c % 2], axis=0, keepdims=True)
            acc = part if acc is None else jnp.maximum(acc, part)

        gather_ref[pl.ds(my_pos, 1), :] = acc

        pl.semaphore_wait(barrier_sem, N_DEV - 1)

        sends = []
        for k in range(1, N_DEV):
            tgt = (my_pos + k) % N_DEV
            rdma = pltpu.make_async_remote_copy(
                src_ref=gather_ref.at[pl.ds(my_pos, 1)],
                dst_ref=gather_ref.at[pl.ds(my_pos, 1)],
                send_sem=send_sems.at[k],
                recv_sem=recv_sems.at[my_pos],
                device_id=(tgt,),
                device_id_type=pl.DeviceIdType.MESH,
            )
            rdma.start()
            sends.append(rdma)

        for k in range(1, N_DEV):
            src = (my_pos + k) % N_DEV
            recv = pltpu.make_async_remote_copy(
                src_ref=gather_ref.at[pl.ds(src, 1)],
                dst_ref=gather_ref.at[pl.ds(src, 1)],
                send_sem=send_sems.at[k],
                recv_sem=recv_sems.at[src],
                device_id=(src,),
                device_id_type=pl.DeviceIdType.MESH,
            )
            recv.wait_recv()
        for rdma in sends:
            rdma.wait_send()

        out_ref[:, :] = jnp.max(gather_ref[:, :], axis=0, keepdims=True)

    return pl.pallas_call(
        body,
        out_shape=jax.ShapeDtypeStruct((1, n), x.dtype),
        in_specs=[pl.BlockSpec(memory_space=pl.ANY)],
        out_specs=pl.BlockSpec(memory_space=pltpu.VMEM),
        scratch_shapes=[
            pltpu.VMEM((2, chunk, n), x.dtype),
            pltpu.VMEM((N_DEV, n), x.dtype),
            pltpu.SemaphoreType.DMA((2,)),
            pltpu.SemaphoreType.DMA((N_DEV,)),
            pltpu.SemaphoreType.DMA((N_DEV,)),
        ],
        compiler_params=pltpu.CompilerParams(collective_id=0),
    )(x)
